# baseline (speedup 1.0000x reference)
import functools

import jax
import jax.numpy as jnp
from jax.experimental import pallas as pl
from jax.experimental.pallas import tpu as pltpu

D_MODEL = 768
N_HEAD = 12
D_HEAD = D_MODEL // N_HEAD
SEQ = 2048
_SCALE = 1.0 / (D_HEAD ** 0.5)
_CH = 512
_NCH = SEQ // _CH


def _dot_t(a, b):
    return jax.lax.dot_general(a, b, (((1,), (1,)), ((), ())),
                               preferred_element_type=jnp.float32)


def _dot(a, b):
    return jax.lax.dot_general(a, b, (((1,), (0,)), ((), ())),
                               preferred_element_type=jnp.float32)


def _kernel(q_ref, kv_ref, wq_ref, wkv_ref, wct_ref, bias_ref, out_ref,
            k_scr, v1_scr, kv1_scr, rkv_scr, qh_scr, lo_scr):
    t = pl.program_id(0)
    slot = jax.lax.rem(t, 2)
    prev = 1 - slot

    @pl.when(t == 0)
    def _proj_kv():
        kvp = _dot_t(kv_ref[...], wkv_ref[...])
        kk = kvp[:, :D_HEAD]
        vv = kvp[:, D_HEAD:]
        kn = jnp.sqrt(jnp.sum(kk * kk, axis=0, keepdims=True))
        vn = jnp.sqrt(jnp.sum(vv * vv, axis=0, keepdims=True))
        kk = kk / jnp.maximum(kn, 1e-12)
        vv = vv / jnp.maximum(vn, 1e-12)
        ones = jnp.ones((SEQ, 1), jnp.float32)
        k_scr[...] = kk
        v1_scr[...] = jnp.concatenate([vv, ones], axis=1)
        kv1_scr[...] = jnp.concatenate([kk, vv, ones], axis=1)

    gate = jax.nn.sigmoid(bias_ref[...])

    qhp = qh_scr[prev]
    pr = jnp.zeros((SEQ, D_HEAD + 1), jnp.float32)
    for c in range(_NCH):
        rkvc = rkv_scr[prev, c * _CH:(c + 1) * _CH, :]
        s2c = _dot_t(qhp, rkvc[:, :D_HEAD])
        p2c = jnp.exp(s2c * _SCALE)
        pr = pr + _dot(p2c, rkvc[:, D_HEAD:])
    r_out = pr[:, :D_HEAD] / pr[:, D_HEAD:]
    out_h = r_out * gate + lo_scr[prev]
    contrib = _dot(out_h, wct_ref[...])

    @pl.when(t == 1)
    def _init():
        out_ref[...] = contrib

    @pl.when(t > 1)
    def _acc():
        out_ref[...] += contrib

    qh = _dot_t(q_ref[...], wq_ref[...])
    s = _dot_t(qh, k_scr[...])
    m = jnp.max(s, axis=1, keepdims=True)
    rkv = jnp.zeros((SEQ, 2 * D_HEAD + 1), jnp.float32)
    pv = jnp.zeros((SEQ, D_HEAD + 1), jnp.float32)
    for c in range(_NCH):
        sc = s[:, c * _CH:(c + 1) * _CH]
        ohc = (sc >= m).astype(jnp.float32)
        pc = jnp.exp(sc * _SCALE)
        rkv = rkv + _dot(ohc, kv1_scr[c * _CH:(c + 1) * _CH, :])
        pv = pv + _dot(pc, v1_scr[c * _CH:(c + 1) * _CH, :])
    local_out = pv[:, :D_HEAD] / pv[:, D_HEAD:]
    qh_scr[slot] = qh
    rkv_scr[slot] = rkv
    lo_scr[slot] = local_out * (1.0 - gate)


@functools.partial(jax.jit, static_argnames=())
def kernel(q, kv, w_q, w_kv, w_concat, bias):
    b, l, dm = q.shape
    q2 = q.reshape(l, dm)
    kv2 = kv.reshape(l, dm)
    wct = w_concat.T
    bias2 = bias.reshape(1, D_HEAD)

    out = pl.pallas_call(
        _kernel,
        grid=(N_HEAD + 1,),
        in_specs=[
            pl.BlockSpec((l, dm), lambda t: (0, 0)),
            pl.BlockSpec((l, dm), lambda t: (0, 0)),
            pl.BlockSpec((D_HEAD, dm), lambda t: (jnp.minimum(t, N_HEAD - 1), 0)),
            pl.BlockSpec((2 * D_HEAD, dm), lambda t: (0, 0)),
            pl.BlockSpec((D_HEAD, dm), lambda t: (jnp.maximum(t - 1, 0), 0)),
            pl.BlockSpec((1, D_HEAD), lambda t: (0, 0)),
        ],
        out_specs=pl.BlockSpec((l, dm), lambda t: (0, 0)),
        out_shape=jax.ShapeDtypeStruct((l, dm), jnp.float32),
        scratch_shapes=[
            pltpu.VMEM((l, D_HEAD), jnp.float32),
            pltpu.VMEM((l, D_HEAD + 1), jnp.float32),
            pltpu.VMEM((l, 2 * D_HEAD + 1), jnp.float32),
            pltpu.VMEM((2, l, 2 * D_HEAD + 1), jnp.float32),
            pltpu.VMEM((2, l, D_HEAD), jnp.float32),
            pltpu.VMEM((2, l, D_HEAD), jnp.float32),
        ],
        compiler_params=pltpu.CompilerParams(
            dimension_semantics=("arbitrary",),
        ),
    )(q2, kv2, w_q, w_kv, wct, bias2)
    return out.reshape(b, l, dm)
